# baseline (speedup 1.0000x reference)
import jax
import jax.numpy as jnp
from jax.experimental import pallas as pl
from jax.experimental.pallas import tpu as pltpu

_N, _D = 2048, 768
_H, _DH = 12, 64
_C = 500
_CP = 512
_K = 64
_DEPTH = 8
_BLK = 256
_NBLK = _N // _BLK
_NV = 256

_pc = pl.pallas_call


def _fused_body(x_ref, fn_ref, w_ref, b_ref, p_ref, pos_ref, gw_ref, eye_ref,
                wo_ref, bo_ref, o_ref,
                q_scr, k_scr, v_scr, mcol_scr, mrow_scr, t_scr, c_scr):
    i = pl.program_id(0)

    @pl.when(i == 0)
    def _centroids():
        p = p_ref[...]
        c = ((((p[:, 0, :] + p[:, 1, :]) + p[:, 2, :]) + p[:, 3, :])
             + p[:, 4, :]) / 5.0
        c_scr[:_C, :] = c
        c_scr[_C:, :] = jnp.zeros((_CP - _C, _D), jnp.float32)

    @pl.when(i < _NBLK)
    def _prep():
        base = i * _BLK
        x = x_ref[...]
        qkv = jax.lax.dot_general(x, w_ref[...], (((1,), (1,)), ((), ())),
                                  preferred_element_type=jnp.float32)
        qkv = qkv + b_ref[...]
        lane64 = jax.lax.broadcasted_iota(jnp.int32, (_BLK, _DH), 1)
        onescol = jnp.where(lane64 == 0, 1.0, 0.0).astype(jnp.bfloat16)
        for h in range(_H):
            q_scr[h, pl.ds(base, _BLK), :] = (
                qkv[:, h * _DH:(h + 1) * _DH] * 0.125).astype(jnp.bfloat16)
            k_scr[h, pl.ds(base, _BLK), :] = (
                qkv[:, _D + h * _DH:_D + (h + 1) * _DH]).astype(jnp.bfloat16)
            v_scr[h, pl.ds(base, _BLK), 0:_DH] = (
                qkv[:, 2 * _D + h * _DH:2 * _D + (h + 1) * _DH]
            ).astype(jnp.bfloat16)
            v_scr[h, pl.ds(base, _BLK), _DH:2 * _DH] = onescol
        sims = jax.lax.dot_general(fn_ref[...], c_scr[...],
                                   (((1,), (1,)), ((), ())),
                                   preferred_element_type=jnp.float32)
        lane = jax.lax.broadcasted_iota(jnp.int32, (_BLK, _CP), 1)
        sims = jnp.where(lane < _C, sims, -1e30)
        nearest = jnp.max(sims, axis=1, keepdims=True)
        gd = 1.0 - nearest
        gw = gw_ref[0, 0]
        pos = pos_ref[0]
        xc = jnp.clip(pos * (1.0 - gw) + gd * gw, 1e-6, 1.0 - 1e-6)
        m = jnp.zeros(xc.shape, jnp.float32)
        for t in range(_DEPTH):
            xs = xc * 3.0
            digit = jnp.floor(xs)
            xc = xs - digit
            m = m + jnp.where(digit == 2.0, float(1 << (_DEPTH - 1 - t)), 0.0)
        mcol_scr[pl.ds(base, _BLK), :] = m
        mrow_scr[0:1, pl.ds(base, _BLK)] = jax.lax.dot_general(
            m, eye_ref[...], (((0,), (0,)), ((), ())),
            preferred_element_type=jnp.float32)

    @pl.when(i == _NBLK)
    def _build_table():
        val = jax.lax.broadcasted_iota(jnp.int32, (1, _NV), 1).astype(jnp.float32)
        mjc = mcol_scr[...]
        colv = jax.lax.broadcasted_iota(jnp.int32, (_N, 1), 0).astype(jnp.float32)
        keyv = jnp.abs(mjc - val) * float(_N) + colv

        def bs_step(_, lh):
            lo, hi = lh
            mid = jnp.floor((lo + hi) * 0.5)
            cnt = jnp.sum(jnp.where(keyv <= mid, 1.0, 0.0), axis=0,
                          keepdims=True)
            pred = cnt >= float(_K)
            return (jnp.where(pred, lo, mid + 1.0), jnp.where(pred, mid, hi))

        lo0 = jnp.zeros((1, _NV), jnp.float32)
        hi0 = jnp.full((1, _NV), float(_N * 256 - 1), jnp.float32)
        _, thr = jax.lax.fori_loop(0, 19, bs_step, (lo0, hi0))
        t_scr[...] = thr

    @pl.when(i >= _NBLK)
    def _attn():
        base = (i - _NBLK) * _BLK
        mi = mcol_scr[pl.ds(base, _BLK), :]
        mj = mrow_scr[...]
        vals = jax.lax.broadcasted_iota(jnp.int32, (_BLK, _NV), 1).astype(jnp.float32)
        onehot = mi == vals
        thr = jnp.sum(jnp.where(onehot, t_scr[...], 0.0), axis=1,
                      keepdims=True)
        col = jax.lax.broadcasted_iota(jnp.int32, (_BLK, _N), 1).astype(jnp.float32)
        key = jnp.abs(mi - mj) * float(_N) + col
        mask = key <= thr

        outs = []
        for h in range(_H):
            qh = q_scr[h, pl.ds(base, _BLK), :]
            s = jax.lax.dot_general(qh, k_scr[h], (((1,), (1,)), ((), ())),
                                    preferred_element_type=jnp.float32)
            p = jnp.exp(jnp.where(mask, s, -1e30)).astype(jnp.bfloat16)
            oha = jax.lax.dot_general(p, v_scr[h], (((1,), (0,)), ((), ())),
                                      preferred_element_type=jnp.float32)
            denom = oha[:, _DH:_DH + 1]
            outs.append(oha[:, :_DH] * (1.0 / denom))
        ob = jnp.concatenate(outs, axis=1)
        res = jax.lax.dot_general(ob, wo_ref[...], (((1,), (1,)), ((), ())),
                                  preferred_element_type=jnp.float32) + bo_ref[...]
        o_ref[...] = res


def kernel(x, shared_pentachora, W_qkv, b_qkv, W_out, b_out, geo_w):
    b, n, d = x.shape
    x2 = x.reshape(n, d)

    pos3 = jnp.linspace(0.0, 1.0, n).reshape(_NBLK, _BLK, 1)
    gw = jax.nn.sigmoid(geo_w).reshape(1, 1)
    eye = jnp.eye(_BLK, dtype=jnp.float32)
    fn = (x / jnp.maximum(jnp.linalg.norm(x, axis=-1, keepdims=True),
                          1e-12)).reshape(n, d)

    def blk_or0(i):
        return (jnp.where(i < _NBLK, i, 0), 0)

    def blk3_or0(i):
        return (jnp.where(i < _NBLK, i, 0), 0, 0)

    out = _pc(
        _fused_body,
        grid=(2 * _NBLK,),
        in_specs=[
            pl.BlockSpec((_BLK, _D), blk_or0),
            pl.BlockSpec((_BLK, _D), blk_or0),
            pl.BlockSpec((3 * _D, _D), lambda i: (0, 0)),
            pl.BlockSpec((1, 3 * _D), lambda i: (0, 0)),
            pl.BlockSpec((_C, 5, _D), lambda i: (0, 0, 0)),
            pl.BlockSpec((1, _BLK, 1), blk3_or0),
            pl.BlockSpec((1, 1), lambda i: (0, 0)),
            pl.BlockSpec((_BLK, _BLK), lambda i: (0, 0)),
            pl.BlockSpec((_D, _D), lambda i: (0, 0)),
            pl.BlockSpec((1, _D), lambda i: (0, 0)),
        ],
        out_specs=pl.BlockSpec(
            (_BLK, _D), lambda i: (jnp.where(i >= _NBLK, i - _NBLK, 0), 0)),
        out_shape=jax.ShapeDtypeStruct((n, d), jnp.float32),
        scratch_shapes=[
            pltpu.VMEM((_H, _N, _DH), jnp.bfloat16),
            pltpu.VMEM((_H, _N, _DH), jnp.bfloat16),
            pltpu.VMEM((_H, _N, 2 * _DH), jnp.bfloat16),
            pltpu.VMEM((_N, 1), jnp.float32),
            pltpu.VMEM((1, _N), jnp.float32),
            pltpu.VMEM((1, _NV), jnp.float32),
            pltpu.VMEM((_CP, _D), jnp.float32),
        ],
    )(x2, fn, W_qkv, b_qkv.reshape(1, 3 * _D), shared_pentachora, pos3, gw,
      eye, W_out, b_out.reshape(1, _D))
    return out.reshape(b, n, d)


# speedup vs baseline: 1.0055x; 1.0055x over previous
import jax
import jax.numpy as jnp
from jax.experimental import pallas as pl
from jax.experimental.pallas import tpu as pltpu

_N, _D = 2048, 768
_H, _DH = 12, 64
_C = 500
_CP = 512
_K = 64
_DEPTH = 8
_BLK = 512
_NBLK = _N // _BLK
_NV = 256

_pc = pl.pallas_call


def _centroid_body(p_ref, c_ref):
    p = p_ref[...]
    c = ((((p[:, 0, :] + p[:, 1, :]) + p[:, 2, :]) + p[:, 3, :]) + p[:, 4, :]) / 5.0
    c_ref[:_C, :] = c
    c_ref[_C:, :] = jnp.zeros((_CP - _C, _D), jnp.float32)


def _fused_body(x_ref, fn_ref, w_ref, b_ref, c_ref, pos_ref, gw_ref, eye_ref,
                wo_ref, bo_ref, o_ref,
                q_scr, k_scr, v_scr, mcol_scr, mrow_scr, t_scr):
    i = pl.program_id(0)

    @pl.when(i < _NBLK)
    def _prep():
        base = i * _BLK
        x = x_ref[...]
        qkv = jax.lax.dot_general(x, w_ref[...], (((1,), (1,)), ((), ())),
                                  preferred_element_type=jnp.float32)
        qkv = qkv + b_ref[...]
        lane64 = jax.lax.broadcasted_iota(jnp.int32, (_BLK, _DH), 1)
        onescol = jnp.where(lane64 == 0, 1.0, 0.0).astype(jnp.bfloat16)
        for h in range(_H):
            q_scr[h, pl.ds(base, _BLK), :] = (
                qkv[:, h * _DH:(h + 1) * _DH] * 0.125).astype(jnp.bfloat16)
            k_scr[h, pl.ds(base, _BLK), :] = (
                qkv[:, _D + h * _DH:_D + (h + 1) * _DH]).astype(jnp.bfloat16)
            v_scr[h, pl.ds(base, _BLK), 0:_DH] = (
                qkv[:, 2 * _D + h * _DH:2 * _D + (h + 1) * _DH]
            ).astype(jnp.bfloat16)
            v_scr[h, pl.ds(base, _BLK), _DH:2 * _DH] = onescol
        sims = jax.lax.dot_general(fn_ref[...], c_ref[...],
                                   (((1,), (1,)), ((), ())),
                                   preferred_element_type=jnp.float32)
        lane = jax.lax.broadcasted_iota(jnp.int32, (_BLK, _CP), 1)
        sims = jnp.where(lane < _C, sims, -1e30)
        nearest = jnp.max(sims, axis=1, keepdims=True)
        gd = 1.0 - nearest
        gw = gw_ref[0, 0]
        pos = pos_ref[0]
        xc = jnp.clip(pos * (1.0 - gw) + gd * gw, 1e-6, 1.0 - 1e-6)
        m = jnp.zeros(xc.shape, jnp.float32)
        for t in range(_DEPTH):
            xs = xc * 3.0
            digit = jnp.floor(xs)
            xc = xs - digit
            m = m + jnp.where(digit == 2.0, float(1 << (_DEPTH - 1 - t)), 0.0)
        mcol_scr[pl.ds(base, _BLK), :] = m
        mrow_scr[0:1, pl.ds(base, _BLK)] = jax.lax.dot_general(
            m, eye_ref[...], (((0,), (0,)), ((), ())),
            preferred_element_type=jnp.float32)

    @pl.when(i == _NBLK)
    def _build_table():
        val = jax.lax.broadcasted_iota(jnp.int32, (1, _NV), 1).astype(jnp.float32)
        mjc = mcol_scr[...]
        colv = jax.lax.broadcasted_iota(jnp.int32, (_N, 1), 0).astype(jnp.float32)
        keyv = jnp.abs(mjc - val) * float(_N) + colv

        def bs_step(_, lh):
            lo, hi = lh
            mid = jnp.floor((lo + hi) * 0.5)
            cnt = jnp.sum(jnp.where(keyv <= mid, 1.0, 0.0), axis=0,
                          keepdims=True)
            pred = cnt >= float(_K)
            return (jnp.where(pred, lo, mid + 1.0), jnp.where(pred, mid, hi))

        lo0 = jnp.zeros((1, _NV), jnp.float32)
        hi0 = jnp.full((1, _NV), float(_N * 256 - 1), jnp.float32)
        _, thr = jax.lax.fori_loop(0, 19, bs_step, (lo0, hi0))
        t_scr[...] = thr

    @pl.when(i >= _NBLK)
    def _attn():
        base = (i - _NBLK) * _BLK
        mi = mcol_scr[pl.ds(base, _BLK), :]
        mj = mrow_scr[...]
        vals = jax.lax.broadcasted_iota(jnp.int32, (_BLK, _NV), 1).astype(jnp.float32)
        onehot = mi == vals
        thr = jnp.sum(jnp.where(onehot, t_scr[...], 0.0), axis=1,
                      keepdims=True)
        col = jax.lax.broadcasted_iota(jnp.int32, (_BLK, _N), 1).astype(jnp.float32)
        key = jnp.abs(mi - mj) * float(_N) + col
        mask = key <= thr

        outs = []
        for h in range(_H):
            qh = q_scr[h, pl.ds(base, _BLK), :]
            s = jax.lax.dot_general(qh, k_scr[h], (((1,), (1,)), ((), ())),
                                    preferred_element_type=jnp.float32)
            p = jnp.exp(jnp.where(mask, s, -1e30)).astype(jnp.bfloat16)
            oha = jax.lax.dot_general(p, v_scr[h], (((1,), (0,)), ((), ())),
                                      preferred_element_type=jnp.float32)
            denom = oha[:, _DH:_DH + 1]
            outs.append(oha[:, :_DH] * (1.0 / denom))
        ob = jnp.concatenate(outs, axis=1)
        res = jax.lax.dot_general(ob, wo_ref[...], (((1,), (1,)), ((), ())),
                                  preferred_element_type=jnp.float32) + bo_ref[...]
        o_ref[...] = res


def kernel(x, shared_pentachora, W_qkv, b_qkv, W_out, b_out, geo_w):
    b, n, d = x.shape
    x2 = x.reshape(n, d)

    cpad = _pc(_centroid_body,
               out_shape=jax.ShapeDtypeStruct((_CP, _D), jnp.float32))(
                   shared_pentachora)

    pos3 = jnp.linspace(0.0, 1.0, n).reshape(_NBLK, _BLK, 1)
    gw = jax.nn.sigmoid(geo_w).reshape(1, 1)
    eye = jnp.eye(_BLK, dtype=jnp.float32)
    fn = (x / jnp.maximum(jnp.linalg.norm(x, axis=-1, keepdims=True),
                          1e-12)).reshape(n, d)

    def blk_or0(i):
        return (jnp.where(i < _NBLK, i, 0), 0)

    def blk3_or0(i):
        return (jnp.where(i < _NBLK, i, 0), 0, 0)

    out = _pc(
        _fused_body,
        grid=(2 * _NBLK,),
        in_specs=[
            pl.BlockSpec((_BLK, _D), blk_or0),
            pl.BlockSpec((_BLK, _D), blk_or0),
            pl.BlockSpec((3 * _D, _D), lambda i: (0, 0)),
            pl.BlockSpec((1, 3 * _D), lambda i: (0, 0)),
            pl.BlockSpec((_CP, _D), lambda i: (0, 0)),
            pl.BlockSpec((1, _BLK, 1), blk3_or0),
            pl.BlockSpec((1, 1), lambda i: (0, 0)),
            pl.BlockSpec((_BLK, _BLK), lambda i: (0, 0)),
            pl.BlockSpec((_D, _D), lambda i: (0, 0)),
            pl.BlockSpec((1, _D), lambda i: (0, 0)),
        ],
        out_specs=pl.BlockSpec(
            (_BLK, _D), lambda i: (jnp.where(i >= _NBLK, i - _NBLK, 0), 0)),
        out_shape=jax.ShapeDtypeStruct((n, d), jnp.float32),
        scratch_shapes=[
            pltpu.VMEM((_H, _N, _DH), jnp.bfloat16),
            pltpu.VMEM((_H, _N, _DH), jnp.bfloat16),
            pltpu.VMEM((_H, _N, 2 * _DH), jnp.bfloat16),
            pltpu.VMEM((_N, 1), jnp.float32),
            pltpu.VMEM((1, _N), jnp.float32),
            pltpu.VMEM((1, _NV), jnp.float32),
        ],
    )(x2, fn, W_qkv, b_qkv.reshape(1, 3 * _D), cpad, pos3, gw, eye,
      W_out, b_out.reshape(1, _D))
    return out.reshape(b, n, d)
